# baseline (speedup 1.0000x reference)
import jax
import jax.numpy as jnp
from jax import lax
from jax.experimental import pallas as pl
from jax.experimental.pallas import tpu as pltpu
from jax.experimental.pallas import tpu_sc as plsc

_N = 1000000
_H = 16
_C = 6
_B = 16384

_NC = 2
_NS = 16
_NW = _NC * _NS
_BPW = _B // _NW
_PH = _BPW // 4


def _body(ur_hbm, vr_hbm, w_hbm, u_tab, v_tab, out_hbm,
          ur_v, vr_v, rows_u, rows_v, w_v, out_v, sem_u, sem_v):
  wid = lax.axis_index("s") * _NC + lax.axis_index("c")
  base = wid * _BPW
  pltpu.sync_copy(ur_hbm.at[wid], ur_v)
  pltpu.sync_copy(vr_hbm.at[wid], vr_v)
  pltpu.sync_copy(w_hbm, w_v)

  for phase in range(4):
    p0 = phase * _PH

    def fetch_body(blk, _, p0=p0):
      j0 = blk * 16
      ru_vec = ur_v[pl.ds(p0 + j0, 16)]
      rv_vec = vr_v[pl.ds(p0 + j0, 16)]
      copies = []
      for k in range(16):
        copies.append(pltpu.async_copy(u_tab.at[ru_vec[k]],
                                       rows_u.at[j0 + k], sem_u))
      for c in copies:
        c.wait()
      copies = []
      for k in range(16):
        copies.append(pltpu.async_copy(v_tab.at[rv_vec[k]],
                                       rows_v.at[j0 + k], sem_v))
      for c in copies:
        c.wait()
      return 0

    lax.fori_loop(0, _PH // 16, fetch_body, 0)

    def row_body(j, _, p0=p0):
      u_row = rows_u[j, :]
      v_row = rows_v[j, :]
      acc = w_v[2 * _H, :]
      for h in range(_H):
        acc = acc + u_row[h] * w_v[h, :]
        acc = acc + v_row[h] * w_v[_H + h, :]
      out_v[j, :] = acc
      return 0

    lax.fori_loop(0, _PH, row_body, 0)

    pltpu.sync_copy(out_v, out_hbm.at[pl.ds(base + p0, _PH)])


_sc_fused = pl.kernel(
    _body,
    out_type=jax.ShapeDtypeStruct((_B, _H), jnp.float32),
    mesh=plsc.VectorSubcoreMesh(core_axis_name="c", subcore_axis_name="s"),
    scratch_types=[
        pltpu.VMEM((_BPW,), jnp.int32),
        pltpu.VMEM((_BPW,), jnp.int32),
        pltpu.VMEM((_PH, _H), jnp.float32),
        pltpu.VMEM((_PH, _H), jnp.float32),
        pltpu.VMEM((2 * _H + 1, 16), jnp.float32),
        pltpu.VMEM((_PH, _H), jnp.float32),
        pltpu.SemaphoreType.DMA,
        pltpu.SemaphoreType.DMA,
    ],
)


@jax.jit
def kernel(X_batch, U, V, W, b):
  x0 = X_batch[:, 0].astype(jnp.int32)
  x1 = X_batch[:, 1].astype(jnp.int32)
  ur = x0.reshape(_NW, _BPW)
  vr = x1.reshape(_NW, _BPW)
  wpad = jnp.zeros((2 * _H + 1, 16), jnp.float32)
  wpad = wpad.at[:2 * _H, :_C].set(W)
  wpad = wpad.at[2 * _H, :_C].set(b)
  out16 = _sc_fused(ur, vr, wpad, U, V)
  return out16[:, :_C]
